# Initial kernel scaffold; baseline (speedup 1.0000x reference)
import jax
import jax.numpy as jnp
from jax.experimental import pallas as pl


def kernel(x, edge_index, W1, att_src1, att_dst1, bias1, W2, att_src2, att_dst2, bias2):
    raise NotImplementedError("write your pallas kernel here")


# speedup vs baseline: 1.0925x; 1.0925x over previous
import jax
import jax.numpy as jnp
from jax.experimental import pallas as pl

N = 10000
E = 320000
D = 128
H = 8
C = 128
NC = 40


def _softmax_body(h_ref, o_ref):
    h = h_ref[...]
    m = jnp.max(h, axis=1, keepdims=True)
    e = jnp.exp(h - m)
    o_ref[...] = e / jnp.sum(e, axis=1, keepdims=True)


def _gat_conv(x, edge_index, W, att_src, att_dst, bias, heads, out_ch):
    n = x.shape[0]
    loop = jnp.arange(n, dtype=edge_index.dtype)
    src = jnp.concatenate([edge_index[0], loop])
    dst = jnp.concatenate([edge_index[1], loop])
    h = (x @ W).reshape(n, heads, out_ch)
    a_src = (h * att_src).sum(-1)
    a_dst = (h * att_dst).sum(-1)
    alpha = a_src[src] + a_dst[dst]
    alpha = jax.nn.leaky_relu(alpha, 0.2)
    ex = jnp.exp(alpha)
    den = jax.ops.segment_sum(ex, dst, num_segments=n)
    msg = h[src] * (ex / (den[dst] + 1e-16))[..., None]
    out = jax.ops.segment_sum(msg, dst, num_segments=n)
    out = out.reshape(n, heads * out_ch)
    return out + bias


def kernel(x, edge_index, W1, att_src1, att_dst1, bias1, W2, att_src2, att_dst2, bias2):
    h = _gat_conv(x, edge_index, W1, att_src1, att_dst1, bias1, H, C)
    h = jax.nn.elu(h)
    h = _gat_conv(h, edge_index, W2, att_src2, att_dst2, bias2, 1, NC)
    return pl.pallas_call(
        _softmax_body,
        out_shape=jax.ShapeDtypeStruct((N, NC), jnp.float32),
    )(h)


# speedup vs baseline: 14.2289x; 13.0237x over previous
import functools

import jax
import jax.numpy as jnp
from jax import lax
from jax.experimental import pallas as pl
from jax.experimental.pallas import tpu as pltpu
from jax.experimental.pallas import tpu_sc as plsc

N = 10000
E = 320000
D = 128
H = 8
C = 128
NC = 40
NP = 10240
NSC = 2
NTEC = 16
NW = NSC * NTEC
EPW = E // NW
KB = 16
ROWS_PER_TEC = NP // NTEC
L2W = 128


def _sc_layer2_edges(src, dst, a2s, a2d, tab2):
    mesh = plsc.VectorSubcoreMesh(core_axis_name="c", subcore_axis_name="s",
                                  num_cores=NSC, num_subcores=NTEC)
    zacc = jnp.zeros((NP, L2W), jnp.float32)

    @functools.partial(
        pl.kernel,
        out_type=jax.ShapeDtypeStruct((NSC, NP, L2W), jnp.float32),
        mesh=mesh,
        compiler_params=pltpu.CompilerParams(needs_layout_passes=False),
        scratch_types=[
            pltpu.VMEM((EPW,), jnp.int32),
            pltpu.VMEM((EPW,), jnp.int32),
            pltpu.VMEM((NP,), jnp.float32),
            pltpu.VMEM((NP,), jnp.float32),
            pltpu.VMEM((KB, L2W), jnp.float32),
            pltpu.VMEM((KB,), jnp.int32),
            pltpu.VMEM_SHARED((NP, L2W), jnp.float32),
            pltpu.VMEM_SHARED((NTEC * KB,), jnp.int32),
            pltpu.SemaphoreType.DMA,
        ],
    )
    def k(src_hbm, dst_hbm, a2s_hbm, a2d_hbm, tab_hbm, zacc_hbm,
          acc_out,
          src_v, dst_v, a2s_v, a2d_v, rows_v, idx_v, acc_sh, idx_sh, sem):
        cid = lax.axis_index("c")
        sid = lax.axis_index("s")
        wid = sid * NSC + cid
        r0 = sid * ROWS_PER_TEC
        pltpu.sync_copy(zacc_hbm.at[pl.ds(r0, ROWS_PER_TEC)],
                        acc_sh.at[pl.ds(r0, ROWS_PER_TEC)])
        pltpu.sync_copy(src_hbm.at[pl.ds(wid * EPW, EPW)], src_v)
        pltpu.sync_copy(dst_hbm.at[pl.ds(wid * EPW, EPW)], dst_v)
        pltpu.sync_copy(a2s_hbm, a2s_v)
        pltpu.sync_copy(a2d_hbm, a2d_v)
        plsc.subcore_barrier()

        def body(bi, carry):
            base = bi * KB
            src16 = src_v[pl.ds(base, KB)]
            dst16 = dst_v[pl.ds(base, KB)]
            asv = plsc.load_gather(a2s_v, [src16])
            adv = plsc.load_gather(a2d_v, [dst16])
            al = asv + adv
            al = jnp.maximum(al, al * 0.2)
            ex = jnp.exp(al)
            pltpu.sync_copy(dst_v.at[pl.ds(base, KB)],
                            idx_sh.at[pl.ds(sid * KB, KB)])
            pltpu.sync_copy(idx_sh.at[pl.ds(sid * KB, KB)], idx_v)
            pltpu.async_copy(tab_hbm.at[src16], rows_v, sem).wait()
            for e in range(KB):
                s = ex[e]
                for j in range(L2W // 16):
                    rows_v[e, pl.ds(j * 16, 16)] = rows_v[e, pl.ds(j * 16, 16)] * s
            pltpu.sync_copy(rows_v, acc_sh.at[idx_v], add=True)
            return carry

        lax.fori_loop(0, EPW // KB, body, 0)
        plsc.subcore_barrier()
        pltpu.sync_copy(acc_sh.at[pl.ds(r0, ROWS_PER_TEC)],
                        acc_out.at[cid, pl.ds(r0, ROWS_PER_TEC)])

    return k(src, dst, a2s, a2d, tab2, zacc)


L1TW = 1152
R1 = 512
NPASS = NP // R1
RD = 128
EPT = E // NTEC
SELCAP = 6560
_SHIFT8 = None


NB1 = 9


def _sc_layer1_edges(src, dst, tabs, a1d_flat):
    mesh = plsc.VectorSubcoreMesh(core_axis_name="c", subcore_axis_name="s",
                                  num_cores=NSC, num_subcores=NTEC)
    zacc = jnp.zeros((R1 + RD, 128), jnp.float32)

    @functools.partial(
        pl.kernel,
        out_type=jax.ShapeDtypeStruct((NB1, NP, 128), jnp.float32),
        mesh=mesh,
        compiler_params=pltpu.CompilerParams(needs_layout_passes=False),
        scratch_types=[
            pltpu.VMEM((EPT,), jnp.int32),
            pltpu.VMEM((EPT,), jnp.int32),
            pltpu.VMEM((SELCAP + 32,), jnp.int32),
            pltpu.VMEM((SELCAP + 32,), jnp.int32),
            pltpu.VMEM(((R1 + RD) * 8,), jnp.float32),
            [pltpu.VMEM((KB, 128), jnp.float32) for _ in range(NB1)],
            pltpu.VMEM((KB,), jnp.int32),
            [pltpu.VMEM_SHARED((R1 + RD, 128), jnp.float32) for _ in range(NB1)],
            pltpu.VMEM_SHARED((NTEC * KB,), jnp.int32),
            pltpu.SemaphoreType.DMA,
        ],
    )
    def k(src_hbm, dst_hbm, tab_hbm, a1d_hbm, zacc_hbm,
          acc_out,
          src_v, dst_v, selsrc, seldst, a1d_v, blks_v, idx_v, accs_sh,
          idx_sh, sem):
        cid = lax.axis_index("c")
        sid = lax.axis_index("s")
        zrows = (R1 + RD) // NTEC
        orows = R1 // NTEC
        pltpu.sync_copy(src_hbm.at[pl.ds(sid * EPT, EPT)], src_v)
        pltpu.sync_copy(dst_hbm.at[pl.ds(sid * EPT, EPT)], dst_v)

        def do_pass(pi, carry0):
            p = pi * NSC + cid
            r0 = p * R1
            for j in range(NB1):
                pltpu.sync_copy(zacc_hbm.at[pl.ds(sid * zrows, zrows)],
                                accs_sh[j].at[pl.ds(sid * zrows, zrows)])
            pltpu.sync_copy(a1d_hbm.at[pl.ds(r0 * 8, (R1 + RD) * 8)], a1d_v)
            plsc.subcore_barrier()

            def scan(bi, nsel):
                base = bi * KB
                s16 = src_v[pl.ds(base, KB)]
                d16 = dst_v[pl.ds(base, KB)]
                dl = d16 - r0
                m = (dl >= 0) & (dl < R1)
                plsc.store_compressed(selsrc.at[pl.ds(nsel, KB)], s16, mask=m)
                plsc.store_compressed(seldst.at[pl.ds(nsel, KB)], dl, mask=m)
                cnt = plsc.all_reduce_population_count(m)
                if cnt.ndim:
                    cnt = cnt[0]
                return nsel + cnt

            nsel = lax.fori_loop(0, EPT // KB, scan, 0)
            selsrc[pl.ds(nsel, KB)] = jnp.zeros((KB,), jnp.int32)
            seldst[pl.ds(nsel, KB)] = jnp.full((KB,), R1, jnp.int32)
            nb = (nsel + KB - 1) // KB

            def pbody(bi, carry):
                base = bi * KB
                s16 = selsrc[pl.ds(base, KB)]
                pltpu.sync_copy(seldst.at[pl.ds(base, KB)],
                                idx_sh.at[pl.ds(sid * KB, KB)])
                pltpu.sync_copy(idx_sh.at[pl.ds(sid * KB, KB)], idx_v)
                cps = [pltpu.async_copy(tab_hbm[j].at[s16], blks_v[j], sem)
                       for j in range(NB1)]
                for cp in cps:
                    cp.wait()

                def edge(e, c2):
                    de = seldst[pl.ds(base + e, KB)][0]
                    gidx = de * 8 + lax.iota(jnp.int32, KB)
                    adv = plsc.load_gather(a1d_v, [gidx])
                    v64 = blks_v[8][e, pl.ds(0, 16)]
                    shf = 8 + (lax.iota(jnp.int32, KB) & 7)
                    asv = lax.gather(
                        v64, shf[:, None],
                        lax.GatherDimensionNumbers(
                            offset_dims=(), collapsed_slice_dims=(0,),
                            start_index_map=(0,)),
                        slice_sizes=(1,),
                        mode=lax.GatherScatterMode.PROMISE_IN_BOUNDS)
                    al = asv + adv
                    al = jnp.maximum(al, al * 0.2)
                    ex = jnp.exp(al)
                    for hh in range(H):
                        s = ex[hh]
                        for j in range(C // 16):
                            col = j * 16
                            blks_v[hh][e, pl.ds(col, 16)] = (
                                blks_v[hh][e, pl.ds(col, 16)] * s)
                    blks_v[8][e, pl.ds(0, 16)] = v64 * ex
                    return c2

                lax.fori_loop(0, KB, edge, 0)
                for j in range(NB1):
                    pltpu.sync_copy(blks_v[j], accs_sh[j].at[idx_v], add=True)
                return carry

            lax.fori_loop(0, nb, pbody, 0)
            plsc.subcore_barrier()
            for j in range(NB1):
                pltpu.sync_copy(accs_sh[j].at[pl.ds(sid * orows, orows)],
                                acc_out.at[j, pl.ds(r0 + sid * orows, orows)])
            plsc.subcore_barrier()
            return carry0

        lax.fori_loop(0, NPASS // NSC, do_pass, 0)

    return k(src, dst, tabs, a1d_flat, zacc)


BLK = 256


def _k1_body(x_ref, w1_ref, a1s_ref, a1d_ref, tab_ref, aa_ref):
    h = jnp.dot(x_ref[...], w1_ref[...], preferred_element_type=jnp.float32)
    asrc = jnp.dot(h, a1s_ref[...], preferred_element_type=jnp.float32)
    adst = jnp.dot(h, a1d_ref[...], preferred_element_type=jnp.float32)
    ones = jnp.ones((BLK, 8), jnp.float32)
    zpad = jnp.zeros((BLK, L1TW - 1040), jnp.float32)
    tab_ref[...] = jnp.concatenate([h, ones, asrc, zpad], axis=1)
    zpad2 = jnp.zeros((BLK, 128 - 16), jnp.float32)
    aa_ref[...] = jnp.concatenate([asrc, adst, zpad2], axis=1)


def _tc_layer1_dense(xp, W1, A1s, A1d):
    return pl.pallas_call(
        _k1_body,
        grid=(NP // BLK,),
        in_specs=[
            pl.BlockSpec((BLK, D), lambda i: (i, 0)),
            pl.BlockSpec((D, H * C), lambda i: (0, 0)),
            pl.BlockSpec((H * C, 8), lambda i: (0, 0)),
            pl.BlockSpec((H * C, 8), lambda i: (0, 0)),
        ],
        out_specs=[
            pl.BlockSpec((BLK, L1TW), lambda i: (i, 0)),
            pl.BlockSpec((BLK, 128), lambda i: (i, 0)),
        ],
        out_shape=[
            jax.ShapeDtypeStruct((NP, L1TW), jnp.float32),
            jax.ShapeDtypeStruct((NP, 128), jnp.float32),
        ],
    )(xp, W1, A1s, A1d)


def _k3_body(acc_ref, tab_ref, aa_ref, w2_ref, b1_ref, tab2_ref):
    aa = aa_ref[...]
    al = aa[:, 0:8] + aa[:, 8:16]
    exs = jnp.exp(jnp.maximum(al, al * 0.2))
    outs = []
    for hh in range(H):
        lo = hh * C
        num = acc_ref[hh] + exs[:, hh:hh + 1] * tab_ref[:, lo:lo + C]
        den = acc_ref[8, :, hh:hh + 1] + exs[:, hh:hh + 1]
        outs.append(num / den)
    o1 = jnp.concatenate(outs, axis=1) + b1_ref[...]
    o1 = jnp.where(o1 > 0, o1, jnp.exp(o1) - 1.0)
    h2e = jnp.dot(o1, w2_ref[...], preferred_element_type=jnp.float32)
    col = lax.broadcasted_iota(jnp.int32, (BLK, L2W), 1)
    tab2_ref[...] = jnp.where(col == NC, 1.0, h2e)


def _tc_layer1_epilogue(acc1b, tab1, aa, W2ext, bias1):
    return pl.pallas_call(
        _k3_body,
        grid=(NP // BLK,),
        in_specs=[
            pl.BlockSpec((NB1, BLK, 128), lambda i: (0, i, 0)),
            pl.BlockSpec((BLK, L1TW), lambda i: (i, 0)),
            pl.BlockSpec((BLK, 128), lambda i: (i, 0)),
            pl.BlockSpec((H * C, L2W), lambda i: (0, 0)),
            pl.BlockSpec((1, H * C), lambda i: (0, 0)),
        ],
        out_specs=pl.BlockSpec((BLK, L2W), lambda i: (i, 0)),
        out_shape=jax.ShapeDtypeStruct((NP, L2W), jnp.float32),
    )(acc1b, tab1, aa, W2ext, bias1.reshape(1, H * C))


def _k4_body(acc0_ref, acc1_ref, tab2_ref, b2_ref, out_ref):
    t = tab2_ref[...]
    a2s = t[:, 41:42]
    a2d = t[:, 42:43]
    al = a2s + a2d
    exs = jnp.exp(jnp.maximum(al, al * 0.2))
    accs = acc0_ref[...] + acc1_ref[...]
    num = accs[:, 0:NC] + exs * t[:, 0:NC]
    den = accs[:, NC:NC + 1] + exs
    o = num / den + b2_ref[...]
    m = jnp.max(o, axis=1, keepdims=True)
    e = jnp.exp(o - m)
    out_ref[...] = e / jnp.sum(e, axis=1, keepdims=True)


def _tc_layer2_epilogue(acc_parts, tab2, bias2):
    b2 = bias2.reshape(1, NC)
    return pl.pallas_call(
        _k4_body,
        grid=(NP // BLK,),
        in_specs=[
            pl.BlockSpec((BLK, L2W), lambda i: (i, 0)),
            pl.BlockSpec((BLK, L2W), lambda i: (i, 0)),
            pl.BlockSpec((BLK, L2W), lambda i: (i, 0)),
            pl.BlockSpec((1, NC), lambda i: (0, 0)),
        ],
        out_specs=pl.BlockSpec((BLK, NC), lambda i: (i, 0)),
        out_shape=jax.ShapeDtypeStruct((NP, NC), jnp.float32),
    )(acc_parts[0], acc_parts[1], tab2, b2)


def kernel(x, edge_index, W1, att_src1, att_dst1, bias1, W2, att_src2, att_dst2, bias2):
    src = edge_index[0]
    dst = edge_index[1]
    xp = jnp.pad(x, ((0, NP - N), (0, 0)))
    A1s = jnp.zeros((H * C, 8), jnp.float32)
    A1s = A1s.at[jnp.arange(H * C), jnp.arange(H * C) // C].set(att_src1.reshape(-1))
    A1d = jnp.zeros((H * C, 8), jnp.float32)
    A1d = A1d.at[jnp.arange(H * C), jnp.arange(H * C) // C].set(att_dst1.reshape(-1))
    W2ext = jnp.zeros((H * C, L2W), jnp.float32)
    W2ext = W2ext.at[:, :NC].set(W2)
    W2ext = W2ext.at[:, 41].set(W2 @ att_src2.reshape(NC))
    W2ext = W2ext.at[:, 42].set(W2 @ att_dst2.reshape(NC))

    tab1, aa = _tc_layer1_dense(xp, W1, A1s, A1d)
    a1d_flat = jnp.pad(aa[:, 8:16].reshape(-1), (0, RD * 8))
    tabs = [tab1[:, j * 128:(j + 1) * 128] for j in range(NB1)]
    acc1 = _sc_layer1_edges(src, dst, tabs, a1d_flat)

    tab2 = _tc_layer1_epilogue(acc1, tab1, aa, W2ext, bias1)

    a2sp = tab2[:, 41]
    a2dp = tab2[:, 42]
    acc_parts = _sc_layer2_edges(src, dst, a2sp, a2dp, tab2)

    out = _tc_layer2_epilogue(acc_parts, tab2, bias2)
    return out[:N]
